# Initial kernel scaffold; baseline (speedup 1.0000x reference)
import jax
import jax.numpy as jnp
from jax.experimental import pallas as pl


def kernel(x, edge_index, W1_self, W1_neigh, b1, W2_self, W2_neigh, b2, W_lin, b_lin):
    raise NotImplementedError("write your pallas kernel here")


# speedup vs baseline: 2.7150x; 2.7150x over previous
import functools

import jax
import jax.numpy as jnp
from jax import lax
from jax.experimental import pallas as pl
from jax.experimental.pallas import tpu as pltpu
from jax.experimental.pallas import tpu_sc as plsc

N = 10000
E = 160000
NP = 10240
D_IN = 256
D_HID = 1024
NT = 16
NCORES = 2
LB = 128
NB = 79
EPT = NB * LB
RPT = NP // NT
C1 = D_IN // 128
C2 = D_HID // 128
RB = 1024
GRID = NP // RB


def _sc_agg_body(cc, nch, *refs):
    tab, srcC, dstT, zrow, out, acc, src_v, dst_v, gbuf = refs
    guard = cc * NCORES > nch

    c = lax.axis_index("c")
    s = lax.axis_index("s")
    row0 = s * RPT
    nz = RPT // LB

    pltpu.sync_copy(dstT.at[s], dst_v)

    for j in range(cc):
        cidx = c * cc + j
        live = cidx < nch

        def _prep():
            for z in range(nz):
                pltpu.sync_copy(zrow, acc.at[pl.ds(row0 + z * LB, LB)])
            pltpu.sync_copy(srcC.at[cidx, s], src_v)
        pl.when(live)(_prep) if guard else _prep()
        plsc.subcore_barrier()

        def _accum():
            def _bstep(b, _):
                pltpu.sync_copy(tab.at[src_v.at[b]], gbuf)
                pltpu.sync_copy(gbuf, acc.at[dst_v.at[b]], add=True)
                return 0
            lax.fori_loop(0, NB, _bstep, 0)
        pl.when(live)(_accum) if guard else _accum()
        plsc.subcore_barrier()

        def _flush():
            pltpu.sync_copy(acc.at[pl.ds(row0, RPT)],
                            out.at[cidx, pl.ds(row0, RPT)])
        pl.when(live)(_flush) if guard else _flush()
        plsc.subcore_barrier()


def _make_sc_agg(cc, nch):
    mesh = plsc.VectorSubcoreMesh(core_axis_name="c", subcore_axis_name="s")
    return pl.kernel(
        functools.partial(_sc_agg_body, cc, nch),
        out_type=jax.ShapeDtypeStruct((cc * NCORES, NP, 128), jnp.float32),
        mesh=mesh,
        scratch_types=[
            pltpu.VMEM_SHARED((NP, 128), jnp.float32),
            pltpu.VMEM((NB, LB), jnp.int32),
            pltpu.VMEM((NB, LB), jnp.int32),
            pltpu.VMEM((LB, 128), jnp.float32),
        ],
    )


def _tc_layer1_body(x_ref, agg_ref, deg_ref, ws_ref, wn_ref, b_ref,
                    h_ref, ht_ref):
    invd = 1.0 / jnp.maximum(deg_ref[:, 0:1], 1.0)
    aggn = jnp.concatenate([agg_ref[j] for j in range(C1)], axis=1) * invd
    h = (jnp.dot(x_ref[...], ws_ref[...], preferred_element_type=jnp.float32)
         + jnp.dot(aggn, wn_ref[...], preferred_element_type=jnp.float32)
         + b_ref[...])
    h = jnp.maximum(h, 0.0)
    h_ref[...] = h
    for j in range(C2):
        ht_ref[j] = h[:, j * 128:(j + 1) * 128]


def _tc_layer2_body(h_ref, agg_ref, deg_ref, ws_ref, wn_ref, b_ref,
                    wl_ref, bl_ref, out_ref):
    invd = 1.0 / jnp.maximum(deg_ref[:, 0:1], 1.0)
    aggn = jnp.concatenate([agg_ref[j] for j in range(C2)], axis=1) * invd
    h2 = (jnp.dot(h_ref[...], ws_ref[...], preferred_element_type=jnp.float32)
          + jnp.dot(aggn, wn_ref[...], preferred_element_type=jnp.float32)
          + b_ref[...])
    g = jax.nn.gelu(h2)
    out_ref[...] = (jnp.dot(g, wl_ref[...], preferred_element_type=jnp.float32)
                    + bl_ref[...])


def _tc_layer1(xp, agg1c, deg16, W1_self, W1_neigh, b1):
    return pl.pallas_call(
        _tc_layer1_body,
        grid=(GRID,),
        in_specs=[
            pl.BlockSpec((RB, D_IN), lambda i: (i, 0)),
            pl.BlockSpec((C1, RB, 128), lambda i: (0, i, 0)),
            pl.BlockSpec((RB, 128), lambda i: (i, 0)),
            pl.BlockSpec((D_IN, D_HID), lambda i: (0, 0)),
            pl.BlockSpec((D_IN, D_HID), lambda i: (0, 0)),
            pl.BlockSpec((1, D_HID), lambda i: (0, 0)),
        ],
        out_specs=[
            pl.BlockSpec((RB, D_HID), lambda i: (i, 0)),
            pl.BlockSpec((C2, RB, 128), lambda i: (0, i, 0)),
        ],
        out_shape=[
            jax.ShapeDtypeStruct((NP, D_HID), jnp.float32),
            jax.ShapeDtypeStruct((C2, NP, 128), jnp.float32),
        ],
    )(xp, agg1c, deg16, W1_self, W1_neigh, b1.reshape(1, D_HID))


def _tc_layer2(h1, agg2c, deg16, W2_self, W2_neigh, b2, W_lin, b_lin):
    return pl.pallas_call(
        _tc_layer2_body,
        grid=(GRID,),
        in_specs=[
            pl.BlockSpec((RB, D_HID), lambda i: (i, 0)),
            pl.BlockSpec((C2, RB, 128), lambda i: (0, i, 0)),
            pl.BlockSpec((RB, 128), lambda i: (i, 0)),
            pl.BlockSpec((D_HID, D_HID), lambda i: (0, 0)),
            pl.BlockSpec((D_HID, D_HID), lambda i: (0, 0)),
            pl.BlockSpec((1, D_HID), lambda i: (0, 0)),
            pl.BlockSpec((D_HID, D_HID), lambda i: (0, 0)),
            pl.BlockSpec((1, D_HID), lambda i: (0, 0)),
        ],
        out_specs=pl.BlockSpec((RB, D_HID), lambda i: (i, 0)),
        out_shape=jax.ShapeDtypeStruct((NP, D_HID), jnp.float32),
    )(h1, agg2c, deg16, W2_self, W2_neigh, b2.reshape(1, D_HID),
      W_lin, b_lin.reshape(1, D_HID))


def kernel(x, edge_index, W1_self, W1_neigh, b1, W2_self, W2_neigh, b2,
           W_lin, b_lin):
    src = edge_index[0].astype(jnp.int32)
    dst = edge_index[1].astype(jnp.int32)
    srcm = jnp.pad(src.reshape(NT, E // NT), ((0, 0), (0, EPT - E // NT)))
    dstm = jnp.pad(dst.reshape(NT, E // NT), ((0, 0), (0, EPT - E // NT)),
                   constant_values=N)
    dstT = dstm.reshape(NT, NB, LB)
    off1 = (jnp.minimum(jnp.arange(C1 + 2, dtype=jnp.int32), C1)
            * NP)[:, None, None]
    src1 = (srcm[None] + off1).reshape(C1 + 2, NT, NB, LB)
    off2 = (jnp.arange(C2, dtype=jnp.int32) * NP)[:, None, None]
    src2 = (srcm[None] + off2).reshape(C2, NT, NB, LB)

    xp = jnp.pad(x, ((0, NP - N), (0, 0)))
    xt = xp.reshape(NP, C1, 128).transpose(1, 0, 2).reshape(C1 * NP, 128)
    tab1 = jnp.concatenate([xt, jnp.ones((NP, 128), jnp.float32)], axis=0)

    zrow = jnp.zeros((LB, 128), jnp.float32)
    agg1out = _make_sc_agg((C1 + 2) // NCORES, C1 + 1)(tab1, src1, dstT, zrow)
    agg1c = agg1out[:C1]
    deg128 = agg1out[C1]
    h1, h1t = _tc_layer1(xp, agg1c, deg128, W1_self, W1_neigh, b1)
    agg2c = _make_sc_agg(C2 // NCORES, C2)(
        h1t.reshape(C2 * NP, 128), src2, dstT, zrow)
    out = _tc_layer2(h1, agg2c, deg128, W2_self, W2_neigh, b2, W_lin, b_lin)
    return out[:N]
